# baseline (speedup 1.0000x reference)
import functools
import math

import jax
import jax.numpy as jnp
from jax.experimental import pallas as pl
from jax.experimental.pallas import tpu as pltpu
from jax.experimental.pallas import tpu_sc as plsc

N_ALL = 10000
N = 2048
NBR = 16
EMB = 128
EDGE = 16
TIME = 32
QD = EMB + TIME
KD = EMB + EDGE + TIME
HEADS = 4
HD = QD // HEADS
HDP = 128
QDP = HEADS * HDP

GW = 128
RBLK = 256
NBLK = N // RBLK

_f32 = jnp.float32
_bf16 = jnp.bfloat16


def _sc_gather(features, nbr_idx, node_idx):
    nidx = nbr_idx.reshape(1, N * NBR).astype(jnp.int32)
    tidx = node_idx.reshape(1, N).astype(jnp.int32)
    mesh = plsc.VectorSubcoreMesh(core_axis_name="c", subcore_axis_name="s")

    @functools.partial(
        pl.kernel,
        out_type=(
            jax.ShapeDtypeStruct((N * NBR, EMB), _f32),
            jax.ShapeDtypeStruct((N, EMB), _f32),
        ),
        mesh=mesh,
    )
    def gather_kernel(feat_hbm, nidx_hbm, tidx_hbm, neigh_hbm, node_hbm):
        def gather_body(i_vmem, o_vmem):
            pltpu.sync_copy(feat_hbm.at[i_vmem.at[0]], o_vmem)

        pltpu.emit_pipeline(
            gather_body,
            grid=(N * NBR // GW,),
            in_specs=[pl.BlockSpec((1, GW), lambda i: (0, i))],
            out_specs=[pl.BlockSpec((GW, EMB), lambda i: (i, 0))],
            core_axis_name=("c", "s"),
            dimension_semantics=(pltpu.PARALLEL,),
        )(nidx_hbm, neigh_hbm)

        pltpu.emit_pipeline(
            gather_body,
            grid=(N // GW,),
            in_specs=[pl.BlockSpec((1, GW), lambda i: (0, i))],
            out_specs=[pl.BlockSpec((GW, EMB), lambda i: (i, 0))],
            core_axis_name=("c", "s"),
            dimension_semantics=(pltpu.PARALLEL,),
        )(tidx_hbm, node_hbm)

    return gather_kernel(features, nidx, tidx)


def _dot_bt(a, b):
    return jax.lax.dot_general(a, b, (((1,), (1,)), ((), ())),
                               preferred_element_type=_f32)


def _store_padded_heads(x160, out_ref, rows):
    out_ref[rows, :] = jnp.zeros((RBLK, QDP), _bf16)
    for h in range(HEADS):
        out_ref[rows, h * HDP:h * HDP + HD] = (
            x160[:, h * HD:(h + 1) * HD].astype(_bf16))


def _mega_body(neigh_hbm, edge_hbm, time_hbm, node_hbm,
               wq, wke, wked, wkt, wve, wved, wvt,
               qb, kb, vb, ones_col, wout, outb, w1a, w1b, b1, w2, b2,
               out_ref,
               neigh_v, edge_v, time_v, node_v, qp_v, kp_v, vp_v, sems):
    scale = 1.0 / math.sqrt(HD)
    srcs = (neigh_hbm, edge_hbm, time_hbm, node_hbm)
    dsts = (neigh_v, edge_v, time_v, node_v)

    def array_copy(a):
        return pltpu.make_async_copy(srcs[a], dsts[a], sems.at[a, 0])

    for a in range(4):
        array_copy(a).start()
    for a in range(4):
        array_copy(a).wait()

    for b in range(NBLK):
        rows = pl.ds(b * RBLK, RBLK)
        nb = neigh_v[rows, :].astype(_bf16)
        eb = edge_v[rows, :].astype(_bf16)
        tb = time_v[rows, :].astype(_bf16)
        ob = node_v[rows, :].astype(_bf16)
        q160 = _dot_bt(ob, wq[...]) * scale + qb[...]
        k160 = (_dot_bt(nb, wke[...]) + _dot_bt(eb, wked[...])
                + _dot_bt(tb, wkt[...]) + kb[...])
        v160 = (_dot_bt(nb, wve[...]) + _dot_bt(eb, wved[...])
                + _dot_bt(tb, wvt[...]) + vb[...])
        _store_padded_heads(q160, qp_v, rows)
        _store_padded_heads(k160, kp_v, rows)
        _store_padded_heads(v160, vp_v, rows)
        vp_v[rows, :] = vp_v[rows, :] + ones_col[...].astype(_bf16)

    dot = functools.partial(jnp.dot, preferred_element_type=_f32)
    for b in range(NBLK):
        rows = pl.ds(b * RBLK, RBLK)
        attn = outb[...]
        for h in range(HEADS):
            cols = pl.ds(h * HDP, HDP)
            qh = qp_v[rows, cols]
            kh = kp_v[:, cols]
            vh = vp_v[:, cols]
            s = _dot_bt(qh, kh)
            e = jnp.exp(s).astype(_bf16)
            ctx_h = dot(e, vh)
            ctx_h = (ctx_h / ctx_h[:, HD:HD + 1]).astype(_bf16)
            attn = attn + dot(ctx_h, wout[h * HDP:(h + 1) * HDP, :])
        hid = jnp.maximum(dot(node_v[rows, :].astype(_bf16), w1a[...])
                          + dot(attn.astype(_bf16), w1b[...]) + b1[...], 0.0)
        out_ref[rows, :] = dot(hid.astype(_bf16), w2[...]) + b2[...]


def kernel(features, edge_feats, time_feats, time_zeros, q_proj_w, k_proj_w,
           v_proj_w, in_proj_b, out_proj_w, out_proj_b, W1, b1, W2, b2,
           neighbor_idx, node_idx):
    neigh_rows, node_emb = _sc_gather(features, neighbor_idx, node_idx)
    neigh_flat = neigh_rows.reshape(N, NBR * EMB)
    edge_flat = edge_feats.reshape(N, NBR * EDGE)
    time_flat = time_feats.reshape(N, NBR * TIME)

    bq = in_proj_b[:QD]
    bk = in_proj_b[QD:2 * QD]
    bv = in_proj_b[2 * QD:]
    qb = (bq + (time_zeros @ q_proj_w[:, EMB:].T)[0])[None, :]

    wq = q_proj_w[:, :EMB].astype(_bf16)

    def split_kv(w):
        w3 = w.reshape(QD, NBR, KD)
        w_emb = w3[:, :, :EMB].reshape(QD, NBR * EMB).astype(_bf16)
        w_edge = w3[:, :, EMB:EMB + EDGE].reshape(QD, NBR * EDGE).astype(_bf16)
        w_time = w3[:, :, EMB + EDGE:].reshape(QD, NBR * TIME).astype(_bf16)
        return w_emb, w_edge, w_time

    wke, wked, wkt = split_kv(k_proj_w)
    wve, wved, wvt = split_kv(v_proj_w)
    ones_col = (((jnp.arange(QDP) % HDP) == HD)[None, :]).astype(_f32)

    wout_p = jnp.pad(out_proj_w.T.reshape(HEADS, HD, QD),
                     ((0, 0), (0, HDP - HD), (0, 0))).reshape(QDP, QD)
    wout_p = wout_p.astype(_bf16)
    w1a = W1[:, :EMB].T.astype(_bf16)
    w1b = W1[:, EMB:].T.astype(_bf16)
    w2t = W2.T.astype(_bf16)

    any_spec = pl.BlockSpec(memory_space=pl.ANY)
    vmem = lambda: pl.BlockSpec(memory_space=pltpu.VMEM)
    out = pl.pallas_call(
        _mega_body,
        in_specs=[any_spec] * 4 + [vmem() for _ in range(18)],
        out_specs=vmem(),
        out_shape=jax.ShapeDtypeStruct((N, EMB), _f32),
        scratch_shapes=[
            pltpu.VMEM((N, NBR * EMB), _f32),
            pltpu.VMEM((N, NBR * EDGE), _f32),
            pltpu.VMEM((N, NBR * TIME), _f32),
            pltpu.VMEM((N, EMB), _f32),
            pltpu.VMEM((N, QDP), _bf16),
            pltpu.VMEM((N, QDP), _bf16),
            pltpu.VMEM((N, QDP), _bf16),
            pltpu.SemaphoreType.DMA((4, NBLK)),
        ],
    )(neigh_flat, edge_flat, time_flat, node_emb,
      wq, wke, wked, wkt, wve, wved, wvt,
      qb, bk[None, :], bv[None, :], ones_col,
      wout_p, out_proj_b[None, :], w1a, w1b, b1[None, :], w2t, b2[None, :])
    return out


# speedup vs baseline: 1.0547x; 1.0547x over previous
import functools
import math

import jax
import jax.numpy as jnp
from jax.experimental import pallas as pl
from jax.experimental.pallas import tpu as pltpu
from jax.experimental.pallas import tpu_sc as plsc

N_ALL = 10000
N = 2048
NBR = 16
EMB = 128
EDGE = 16
TIME = 32
QD = EMB + TIME
KD = EMB + EDGE + TIME
HEADS = 4
HD = QD // HEADS
HDP = 128
QDP = HEADS * HDP

GW = 128
RBLK = 256
NBLK = N // RBLK

_f32 = jnp.float32
_bf16 = jnp.bfloat16


def _sc_gather(features, nbr_idx, node_idx):
    nidx = nbr_idx.reshape(1, N * NBR).astype(jnp.int32)
    tidx = node_idx.reshape(1, N).astype(jnp.int32)
    mesh = plsc.VectorSubcoreMesh(core_axis_name="c", subcore_axis_name="s")

    @functools.partial(
        pl.kernel,
        out_type=(
            jax.ShapeDtypeStruct((N * NBR, EMB), _f32),
            jax.ShapeDtypeStruct((N, EMB), _f32),
        ),
        mesh=mesh,
    )
    def gather_kernel(feat_hbm, nidx_hbm, tidx_hbm, neigh_hbm, node_hbm):
        def gather_body(i_vmem, o_vmem):
            pltpu.sync_copy(feat_hbm.at[i_vmem.at[0]], o_vmem)

        pltpu.emit_pipeline(
            gather_body,
            grid=(N * NBR // GW,),
            in_specs=[pl.BlockSpec((1, GW), lambda i: (0, i))],
            out_specs=[pl.BlockSpec((GW, EMB), lambda i: (i, 0))],
            core_axis_name=("c", "s"),
            dimension_semantics=(pltpu.PARALLEL,),
        )(nidx_hbm, neigh_hbm)

        pltpu.emit_pipeline(
            gather_body,
            grid=(N // GW,),
            in_specs=[pl.BlockSpec((1, GW), lambda i: (0, i))],
            out_specs=[pl.BlockSpec((GW, EMB), lambda i: (i, 0))],
            core_axis_name=("c", "s"),
            dimension_semantics=(pltpu.PARALLEL,),
        )(tidx_hbm, node_hbm)

    return gather_kernel(features, nidx, tidx)


def _dot_bt(a, b):
    return jax.lax.dot_general(a, b, (((1,), (1,)), ((), ())),
                               preferred_element_type=_f32)


def _store_padded_heads(x160, out_ref, rows):
    out_ref[rows, :] = jnp.zeros((RBLK, QDP), _bf16)
    for h in range(HEADS):
        out_ref[rows, h * HDP:h * HDP + HD] = (
            x160[:, h * HD:(h + 1) * HD].astype(_bf16))


def _mega_body(neigh_hbm, edge_hbm, time_hbm, node_hbm,
               wq, wke, wked, wkt, wve, wved, wvt,
               qb, kb, vb, ones_col, wout, outb, w1a, w1b, b1, w2, b2,
               out_ref,
               neigh_v, edge_v, time_v, node_v, qp_v, kpart_v, vpart_v,
               acc_v, kc_v, vc_v, sems):
    dot = functools.partial(jnp.dot, preferred_element_type=_f32)
    scale = 1.0 / math.sqrt(HD)

    small = [
        pltpu.make_async_copy(node_hbm, node_v, sems.at[0]),
        pltpu.make_async_copy(edge_hbm, edge_v, sems.at[1]),
        pltpu.make_async_copy(time_hbm, time_v, sems.at[2]),
    ]
    for c in small:
        c.start()

    def chunk_copy(b):
        rows = pl.ds(b * RBLK, RBLK)
        return pltpu.make_async_copy(
            neigh_hbm.at[rows, :], neigh_v.at[rows, :], sems.at[3 + b])

    for b in range(NBLK):
        chunk_copy(b).start()

    small[0].wait()
    for b in range(NBLK):
        rows = pl.ds(b * RBLK, RBLK)
        q160 = _dot_bt(node_v[rows, :].astype(_bf16), wq[...]) * scale \
            + qb[...]
        _store_padded_heads(q160, qp_v, rows)

    small[1].wait()
    small[2].wait()
    for b in range(NBLK):
        rows = pl.ds(b * RBLK, RBLK)
        eb = edge_v[rows, :].astype(_bf16)
        tb = time_v[rows, :].astype(_bf16)
        kpart_v[rows, :] = (_dot_bt(eb, wked[...]) + _dot_bt(tb, wkt[...])
                            + kb[...])
        vpart_v[rows, :] = (_dot_bt(eb, wved[...]) + _dot_bt(tb, wvt[...])
                            + vb[...])

    acc_v[...] = jnp.zeros(acc_v.shape, _f32)
    for c in range(NBLK):
        krows = pl.ds(c * RBLK, RBLK)
        chunk_copy(c).wait()
        nb = neigh_v[krows, :].astype(_bf16)
        k160 = _dot_bt(nb, wke[...]) + kpart_v[krows, :]
        v160 = _dot_bt(nb, wve[...]) + vpart_v[krows, :]
        all_rows = pl.ds(0, RBLK)
        _store_padded_heads(k160, kc_v, all_rows)
        _store_padded_heads(v160, vc_v, all_rows)
        vc_v[...] = vc_v[...] + ones_col[...].astype(_bf16)
        for h in range(HEADS):
            cols = pl.ds(h * HDP, HDP)
            kh = kc_v[:, h * HDP:(h + 1) * HDP]
            vh = vc_v[:, h * HDP:(h + 1) * HDP]
            for b in range(NBLK):
                qrows = pl.ds(b * RBLK, RBLK)
                s = _dot_bt(qp_v[qrows, cols], kh)
                e = jnp.exp(s).astype(_bf16)
                acc_v[qrows, cols] = acc_v[qrows, cols] + dot(e, vh)

    for b in range(NBLK):
        rows = pl.ds(b * RBLK, RBLK)
        attn = outb[...]
        for h in range(HEADS):
            ctx_h = acc_v[rows, h * HDP:(h + 1) * HDP]
            ctx_h = (ctx_h / ctx_h[:, HD:HD + 1]).astype(_bf16)
            attn = attn + dot(ctx_h, wout[h * HDP:(h + 1) * HDP, :])
        hid = jnp.maximum(dot(node_v[rows, :].astype(_bf16), w1a[...])
                          + dot(attn.astype(_bf16), w1b[...]) + b1[...], 0.0)
        out_ref[rows, :] = dot(hid.astype(_bf16), w2[...]) + b2[...]


def kernel(features, edge_feats, time_feats, time_zeros, q_proj_w, k_proj_w,
           v_proj_w, in_proj_b, out_proj_w, out_proj_b, W1, b1, W2, b2,
           neighbor_idx, node_idx):
    neigh_rows, node_emb = _sc_gather(features, neighbor_idx, node_idx)
    neigh_flat = neigh_rows.reshape(N, NBR * EMB)
    edge_flat = edge_feats.reshape(N, NBR * EDGE)
    time_flat = time_feats.reshape(N, NBR * TIME)

    bq = in_proj_b[:QD]
    bk = in_proj_b[QD:2 * QD]
    bv = in_proj_b[2 * QD:]
    qb = (bq + (time_zeros @ q_proj_w[:, EMB:].T)[0])[None, :]

    wq = q_proj_w[:, :EMB].astype(_bf16)

    def split_kv(w):
        w3 = w.reshape(QD, NBR, KD)
        w_emb = w3[:, :, :EMB].reshape(QD, NBR * EMB).astype(_bf16)
        w_edge = w3[:, :, EMB:EMB + EDGE].reshape(QD, NBR * EDGE).astype(_bf16)
        w_time = w3[:, :, EMB + EDGE:].reshape(QD, NBR * TIME).astype(_bf16)
        return w_emb, w_edge, w_time

    wke, wked, wkt = split_kv(k_proj_w)
    wve, wved, wvt = split_kv(v_proj_w)
    ones_col = (((jnp.arange(QDP) % HDP) == HD)[None, :]).astype(_f32)

    wout_p = jnp.pad(out_proj_w.T.reshape(HEADS, HD, QD),
                     ((0, 0), (0, HDP - HD), (0, 0))).reshape(QDP, QD)
    wout_p = wout_p.astype(_bf16)
    w1a = W1[:, :EMB].T.astype(_bf16)
    w1b = W1[:, EMB:].T.astype(_bf16)
    w2t = W2.T.astype(_bf16)

    any_spec = pl.BlockSpec(memory_space=pl.ANY)
    vmem = lambda: pl.BlockSpec(memory_space=pltpu.VMEM)
    out = pl.pallas_call(
        _mega_body,
        in_specs=[any_spec] * 4 + [vmem() for _ in range(18)],
        out_specs=vmem(),
        out_shape=jax.ShapeDtypeStruct((N, EMB), _f32),
        scratch_shapes=[
            pltpu.VMEM((N, NBR * EMB), _f32),
            pltpu.VMEM((N, NBR * EDGE), _f32),
            pltpu.VMEM((N, NBR * TIME), _f32),
            pltpu.VMEM((N, EMB), _f32),
            pltpu.VMEM((N, QDP), _bf16),
            pltpu.VMEM((N, QD), _f32),
            pltpu.VMEM((N, QD), _f32),
            pltpu.VMEM((N, QDP), _f32),
            pltpu.VMEM((RBLK, QDP), _bf16),
            pltpu.VMEM((RBLK, QDP), _bf16),
            pltpu.SemaphoreType.DMA((3 + NBLK,)),
        ],
    )(neigh_flat, edge_flat, time_flat, node_emb,
      wq, wke, wked, wkt, wve, wved, wvt,
      qb, bk[None, :], bv[None, :], ones_col,
      wout_p, out_proj_b[None, :], w1a, w1b, b1[None, :], w2t, b2[None, :])
    return out


# speedup vs baseline: 1.0667x; 1.0114x over previous
import functools
import math

import jax
import jax.numpy as jnp
from jax.experimental import pallas as pl
from jax.experimental.pallas import tpu as pltpu
from jax.experimental.pallas import tpu_sc as plsc

N_ALL = 10000
N = 2048
NBR = 16
EMB = 128
EDGE = 16
TIME = 32
QD = EMB + TIME
KD = EMB + EDGE + TIME
HEADS = 4
HD = QD // HEADS
HDP = 128
QDP = HEADS * HDP

GW = 128
RBLK = 256
NBLK = N // RBLK

_f32 = jnp.float32
_bf16 = jnp.bfloat16


def _sc_gather(features, nbr_idx, node_idx):
    nidx = nbr_idx.reshape(1, N * NBR).astype(jnp.int32)
    tidx = node_idx.reshape(1, N).astype(jnp.int32)
    mesh = plsc.VectorSubcoreMesh(core_axis_name="c", subcore_axis_name="s")

    @functools.partial(
        pl.kernel,
        out_type=(
            jax.ShapeDtypeStruct((N * NBR, EMB), _f32),
            jax.ShapeDtypeStruct((N, EMB), _f32),
        ),
        mesh=mesh,
    )
    def gather_kernel(feat_hbm, nidx_hbm, tidx_hbm, neigh_hbm, node_hbm):
        def gather_body(i_vmem, o_vmem):
            pltpu.sync_copy(feat_hbm.at[i_vmem.at[0]], o_vmem)

        pltpu.emit_pipeline(
            gather_body,
            grid=(N * NBR // GW,),
            in_specs=[pl.BlockSpec((1, GW), lambda i: (0, i))],
            out_specs=[pl.BlockSpec((GW, EMB), lambda i: (i, 0))],
            core_axis_name=("c", "s"),
            dimension_semantics=(pltpu.PARALLEL,),
        )(nidx_hbm, neigh_hbm)

        pltpu.emit_pipeline(
            gather_body,
            grid=(N // GW,),
            in_specs=[pl.BlockSpec((1, GW), lambda i: (0, i))],
            out_specs=[pl.BlockSpec((GW, EMB), lambda i: (i, 0))],
            core_axis_name=("c", "s"),
            dimension_semantics=(pltpu.PARALLEL,),
        )(tidx_hbm, node_hbm)

    return gather_kernel(features, nidx, tidx)


def _dot_bt(a, b):
    return jax.lax.dot_general(a, b, (((1,), (1,)), ((), ())),
                               preferred_element_type=_f32)


def _store_padded_heads(x160, out_ref, rows):
    out_ref[rows, :] = jnp.zeros((RBLK, QDP), _bf16)
    for h in range(HEADS):
        out_ref[rows, h * HDP:h * HDP + HD] = (
            x160[:, h * HD:(h + 1) * HD].astype(_bf16))


def _mega_body(neigh, edge, time, node,
               wq, wke, wked, wkt, wve, wved, wvt,
               qb, kb, vb, ones_col, wout, outb, w1a, w1b, b1, w2, b2,
               out_ref,
               qp_v, acc_v, kc_v, vc_v):
    dot = functools.partial(jnp.dot, preferred_element_type=_f32)
    scale = 1.0 / math.sqrt(HD)
    c = pl.program_id(0)

    @pl.when(c == 0)
    def _():
        acc_v[...] = jnp.zeros(acc_v.shape, _f32)
        for b in range(NBLK):
            rows = pl.ds(b * RBLK, RBLK)
            q160 = _dot_bt(node[rows, :].astype(_bf16), wq[...]) * scale \
                + qb[...]
            _store_padded_heads(q160, qp_v, rows)

    nb = neigh[...].astype(_bf16)
    eb = edge[...].astype(_bf16)
    tb = time[...].astype(_bf16)
    k160 = (_dot_bt(nb, wke[...]) + _dot_bt(eb, wked[...])
            + _dot_bt(tb, wkt[...]) + kb[...])
    v160 = (_dot_bt(nb, wve[...]) + _dot_bt(eb, wved[...])
            + _dot_bt(tb, wvt[...]) + vb[...])
    all_rows = pl.ds(0, RBLK)
    _store_padded_heads(k160, kc_v, all_rows)
    _store_padded_heads(v160, vc_v, all_rows)
    vc_v[...] = vc_v[...] + ones_col[...].astype(_bf16)
    for h in range(HEADS):
        cols = pl.ds(h * HDP, HDP)
        kh = kc_v[:, h * HDP:(h + 1) * HDP]
        vh = vc_v[:, h * HDP:(h + 1) * HDP]
        for b in range(NBLK):
            qrows = pl.ds(b * RBLK, RBLK)
            s = _dot_bt(qp_v[qrows, cols], kh)
            e = jnp.exp(s).astype(_bf16)
            acc_v[qrows, cols] = acc_v[qrows, cols] + dot(e, vh)

    @pl.when(c == NBLK - 1)
    def _():
        for b in range(NBLK):
            rows = pl.ds(b * RBLK, RBLK)
            attn = outb[...]
            for h in range(HEADS):
                ctx_h = acc_v[rows, h * HDP:(h + 1) * HDP]
                ctx_h = (ctx_h / ctx_h[:, HD:HD + 1]).astype(_bf16)
                attn = attn + dot(ctx_h, wout[h * HDP:(h + 1) * HDP, :])
            hid = jnp.maximum(dot(node[rows, :].astype(_bf16), w1a[...])
                              + dot(attn.astype(_bf16), w1b[...]) + b1[...],
                              0.0)
            out_ref[rows, :] = dot(hid.astype(_bf16), w2[...]) + b2[...]


def kernel(features, edge_feats, time_feats, time_zeros, q_proj_w, k_proj_w,
           v_proj_w, in_proj_b, out_proj_w, out_proj_b, W1, b1, W2, b2,
           neighbor_idx, node_idx):
    neigh_rows, node_emb = _sc_gather(features, neighbor_idx, node_idx)
    neigh_flat = neigh_rows.reshape(N, NBR * EMB)
    edge_flat = edge_feats.reshape(N, NBR * EDGE)
    time_flat = time_feats.reshape(N, NBR * TIME)

    bq = in_proj_b[:QD]
    bk = in_proj_b[QD:2 * QD]
    bv = in_proj_b[2 * QD:]
    qb = (bq + (time_zeros @ q_proj_w[:, EMB:].T)[0])[None, :]

    wq = q_proj_w[:, :EMB].astype(_bf16)

    def split_kv(w):
        w3 = w.reshape(QD, NBR, KD)
        w_emb = w3[:, :, :EMB].reshape(QD, NBR * EMB).astype(_bf16)
        w_edge = w3[:, :, EMB:EMB + EDGE].reshape(QD, NBR * EDGE).astype(_bf16)
        w_time = w3[:, :, EMB + EDGE:].reshape(QD, NBR * TIME).astype(_bf16)
        return w_emb, w_edge, w_time

    wke, wked, wkt = split_kv(k_proj_w)
    wve, wved, wvt = split_kv(v_proj_w)
    ones_col = (((jnp.arange(QDP) % HDP) == HD)[None, :]).astype(_f32)

    wout_p = jnp.pad(out_proj_w.T.reshape(HEADS, HD, QD),
                     ((0, 0), (0, HDP - HD), (0, 0))).reshape(QDP, QD)
    wout_p = wout_p.astype(_bf16)
    w1a = W1[:, :EMB].T.astype(_bf16)
    w1b = W1[:, EMB:].T.astype(_bf16)
    w2t = W2.T.astype(_bf16)

    full = lambda shape: pl.BlockSpec(shape, lambda i: (0, 0))
    chunk = lambda width: pl.BlockSpec((RBLK, width), lambda i: (i, 0))
    out = pl.pallas_call(
        _mega_body,
        grid=(NBLK,),
        in_specs=[
            chunk(NBR * EMB), chunk(NBR * EDGE), chunk(NBR * TIME),
            full((N, EMB)),
            full((QD, EMB)), full((QD, NBR * EMB)), full((QD, NBR * EDGE)),
            full((QD, NBR * TIME)), full((QD, NBR * EMB)),
            full((QD, NBR * EDGE)), full((QD, NBR * TIME)),
            full((1, QD)), full((1, QD)), full((1, QD)), full((1, QDP)),
            full((QDP, QD)), full((1, QD)),
            full((EMB, EMB)), full((QD, EMB)), full((1, EMB)),
            full((EMB, EMB)), full((1, EMB)),
        ],
        out_specs=full((N, EMB)),
        out_shape=jax.ShapeDtypeStruct((N, EMB), _f32),
        scratch_shapes=[
            pltpu.VMEM((N, QDP), _bf16),
            pltpu.VMEM((N, QDP), _f32),
            pltpu.VMEM((RBLK, QDP), _bf16),
            pltpu.VMEM((RBLK, QDP), _bf16),
        ],
    )(neigh_flat, edge_flat, time_flat, node_emb,
      wq, wke, wked, wkt, wve, wved, wvt,
      qb, bk[None, :], bv[None, :], ones_col,
      wout_p, out_proj_b[None, :], w1a, w1b, b1[None, :], w2t, b2[None, :])
    return out
